# baseline (speedup 1.0000x reference)
import functools

import jax
import jax.numpy as jnp
from jax import lax
from jax.experimental import pallas as pl
from jax.experimental.pallas import tpu as pltpu
from jax.experimental.pallas import tpu_sc as plsc

_BATCH = 16384
_D = 100
_HALF = _D // 2
_NW = 32
_MAXF = 4


def _body(xt_hbm, out_hbm, stage, sem_in, sem_out):
    wid = lax.axis_index("s") * 2 + lax.axis_index("c")
    nf = jnp.where(wid < 4, 4, 3)
    j0 = jnp.where(wid < 4, 4 * wid, 16 + 3 * (wid - 4))

    def src_col(i):
        j = j0 + i
        return jnp.where(j < _HALF, 2 * j, 2 * j - (_D - 1))

    for i in range(_MAXF):
        @pl.when(i < nf)
        def _(i=i):
            pltpu.async_copy(xt_hbm.at[src_col(i)], out_hbm.at[j0 + i, 0],
                             sem_out)
    for i in range(_MAXF):
        @pl.when(i < nf)
        def _(i=i):
            pltpu.make_async_copy(xt_hbm.at[0], out_hbm.at[0, 0],
                                  sem_out).wait()


_sc_permute_rows = functools.partial(
    pl.kernel,
    mesh=plsc.VectorSubcoreMesh(core_axis_name="c", subcore_axis_name="s"),
    out_type=jax.ShapeDtypeStruct((_D, 1, _BATCH), jnp.float32),
    scratch_types=[
        pltpu.VMEM((_MAXF, _BATCH), jnp.float32),
        pltpu.SemaphoreType.DMA,
        pltpu.SemaphoreType.DMA,
    ],
    compiler_params=pltpu.CompilerParams(
        needs_layout_passes=False, disable_bounds_checks=True),
)(_body)


def kernel(x):
    yt = _sc_permute_rows(x.T)
    return jnp.transpose(yt, (2, 0, 1))


# speedup vs baseline: 8.8640x; 8.8640x over previous
import functools

import jax
import jax.numpy as jnp
from jax import lax
from jax.experimental import pallas as pl
from jax.experimental.pallas import tpu as pltpu
from jax.experimental.pallas import tpu_sc as plsc

_BATCH = 16384
_D = 100
_HALF = _D // 2
_NW = 32
_MAXF = 4


def _body(xt_hbm, out_hbm, stage, sem_in, sem_out):
    wid = lax.axis_index("s") * 2 + lax.axis_index("c")
    nf = jnp.where(wid < 4, 4, 3)
    j0 = jnp.where(wid < 4, 4 * wid, 16 + 3 * (wid - 4))

    def src_col(i):
        j = j0 + i
        return jnp.where(j < _HALF, 2 * j, 2 * j - (_D - 1))

    for i in range(_MAXF):
        @pl.when(i < nf)
        def _(i=i):
            pltpu.async_copy(xt_hbm.at[src_col(i)], stage.at[i], sem_in.at[i])
    for i in range(_MAXF):
        @pl.when(i < nf)
        def _(i=i):
            pltpu.make_async_copy(xt_hbm.at[0], stage.at[i],
                                  sem_in.at[i]).wait()
            pltpu.async_copy(stage.at[i], out_hbm.at[j0 + i, 0],
                             sem_out.at[i])
    for i in range(_MAXF):
        @pl.when(i < nf)
        def _(i=i):
            pltpu.make_async_copy(stage.at[i], out_hbm.at[0, 0],
                                  sem_out.at[i]).wait()


_sc_permute_rows = functools.partial(
    pl.kernel,
    mesh=plsc.VectorSubcoreMesh(core_axis_name="c", subcore_axis_name="s"),
    out_type=jax.ShapeDtypeStruct((_D, 1, _BATCH), jnp.float32),
    scratch_types=[
        pltpu.VMEM((_MAXF, _BATCH), jnp.float32),
        pltpu.SemaphoreType.DMA((_MAXF,)),
        pltpu.SemaphoreType.DMA((_MAXF,)),
    ],
    compiler_params=pltpu.CompilerParams(
        needs_layout_passes=False, disable_bounds_checks=True),
)(_body)


def kernel(x):
    yt = _sc_permute_rows(x.T)
    return jnp.transpose(yt, (2, 0, 1))
